# baseline (speedup 1.0000x reference)
import functools

import jax
import jax.numpy as jnp
from jax import lax
from jax.experimental import pallas as pl
from jax.experimental.pallas import tpu as pltpu
from jax.experimental.pallas import tpu_sc as plsc

_B, _R = 4, 100
_HW = 784
_C = 81
_CB = _C * _B
_N = 400

_KSC = 112
_KTC = _HW - _KSC
_NW = 16
_PXPT = _KSC // _NW

_PPB = 112
_GTC = _KTC // _PPB
_ROWS = _PPB * _CB
_QR = _PPB * _B
_PCH = 2
_CHR = _PCH * _CB
_CHQ = _PCH * _B

_LN2 = 0.6931471805599453


def _vlog(x):
    bits = lax.bitcast_convert_type(x, jnp.int32)
    e = ((bits >> 23) & 0xFF) - 127
    m = lax.bitcast_convert_type((bits & 0x007FFFFF) | 0x3F800000, jnp.float32)
    z = (m - jnp.float32(1.0)) / (m + jnp.float32(1.0))
    z2 = z * z
    s = jnp.float32(2.0) * z * (
        jnp.float32(1.0)
        + z2 * (jnp.float32(1.0 / 3.0)
                + z2 * (jnp.float32(1.0 / 5.0)
                        + z2 * (jnp.float32(1.0 / 7.0)
                                + z2 * jnp.float32(1.0 / 9.0))))
    )
    return e.astype(jnp.float32) * jnp.float32(_LN2) + s


def _sc_partial(pred_v, tm_v, tci_flat):
    mesh = plsc.VectorSubcoreMesh(core_axis_name="c", subcore_axis_name="s", num_cores=1)

    @functools.partial(
        pl.kernel,
        mesh=mesh,
        out_type=jax.ShapeDtypeStruct((_NW, 16), jnp.float32),
        scratch_types=[
            pltpu.VMEM((_CB + 4, _R), jnp.float32),
            pltpu.VMEM((_CB + 4, _R), jnp.float32),
            pltpu.VMEM((8, _R), jnp.float32),
            pltpu.VMEM((8, _R), jnp.float32),
            pltpu.VMEM((_N,), jnp.int32),
            pltpu.VMEM((16,), jnp.float32),
            pltpu.SemaphoreType.DMA,
            pltpu.SemaphoreType.DMA,
            pltpu.SemaphoreType.DMA,
        ],
    )
    def k(pred_hbm, tm_hbm, tci_hbm, out_hbm, blkA, blkB, tmbA, tmbB,
          cflat, accv, semA, semB, semT):
        wid = lax.axis_index("s")
        pltpu.sync_copy(tci_hbm, cflat)
        iota = lax.iota(jnp.int32, 16)
        tail_m = iota >= 12
        zero16 = jnp.zeros((16,), jnp.float32)
        eps = jnp.float32(1e-7)
        one = jnp.float32(1.0)

        blks = (blkA, blkB)
        tmbs = (tmbA, tmbB)
        sems = (semA, semB)

        def px_of(j):
            return _KTC + wid + j * _NW

        def start(j):
            r0 = px_of(j) * _CB
            off = r0 & 7
            base = pl.multiple_of(r0 - off, 8)
            cp = pltpu.async_copy(
                pred_hbm.at[pl.ds(base, _CB + 4)], blks[j % 2], sems[j % 2])
            t0 = px_of(j) * _B
            toff = t0 & 7
            cpt = pltpu.async_copy(
                tm_hbm.at[pl.ds(pl.multiple_of(t0 - toff, 8), 8)],
                tmbs[j % 2], sems[j % 2])
            return (cp, cpt), (off, toff)

        def compute(j, offs):
            off, toff = offs
            blk = blks[j % 2]
            tmb = tmbs[j % 2]
            pacc = zero16
            for b in range(_B):
                cs_list = [
                    cflat[pl.ds(b * _R + (84 if l == 6 else l * 16), 16)]
                    for l in range(7)
                ]

                def crow(c, acc7):
                    out = []
                    for l in range(7):
                        m = cs_list[l] == c
                        if l == 6:
                            m = m & tail_m
                        xv = blk[off + c * _B + b,
                                 pl.ds(84 if l == 6 else l * 16, 16)]
                        out.append(acc7[l] + jnp.where(m, xv, zero16))
                    return tuple(out)

                acc7 = lax.fori_loop(0, _C, crow, (zero16,) * 7, unroll=2)

                for l in range(7):
                    s = 84 if l == 6 else l * 16
                    p = jnp.minimum(jnp.maximum(acc7[l], eps), one - eps)
                    y = tmb[toff + b, pl.ds(s, 16)]
                    bce = -(y * _vlog(p) + (one - y) * _vlog(one - p))
                    m = cs_list[l] > 0
                    if l == 6:
                        m = m & tail_m
                    pacc = pacc + jnp.where(m, bce, zero16)
            return pacc

        total = zero16
        cps, offs = start(0)
        for j in range(_PXPT):
            for c_ in cps:
                c_.wait()
            cur = offs
            if j + 1 < _PXPT:
                cps, offs = start(j + 1)
            total = total + compute(j, cur)

        accv[...] = total
        pltpu.sync_copy(accv, out_hbm.at[wid])

    return k(pred_v, tm_v, tci_flat)


def _tc_partial(target_class_ids, pred_v, tm_v):
    def body(cls_ref, pred_ref, tm_ref, out_ref, oh_ref, vm_ref, acc_ref):
        g = pl.program_id(0)

        @pl.when(g == 0)
        def _():
            cls = cls_ref[...]
            cid = jax.lax.broadcasted_iota(jnp.int32, (_C, _B, _R), 0)
            oh1 = (cid == cls[None, :, :]).astype(jnp.float32).reshape(_CB, _R)
            vm1 = (cls > 0).astype(jnp.float32)
            for q in range(_PCH):
                oh_ref[pl.ds(q * _CB, _CB), :] = oh1
                vm_ref[pl.ds(q * _B, _B), :] = vm1
            acc_ref[...] = jnp.zeros((_CHQ, _R), jnp.float32)

        eps = jnp.float32(1e-7)
        one = jnp.float32(1.0)
        oh = oh_ref[...]
        vm = vm_ref[...]
        for q in range(_PPB // _PCH):
            xm = pred_ref[pl.ds(q * _CHR, _CHR), :] * oh
            yp = jnp.sum(xm.reshape(_PCH, _C, _B, _R), axis=1)
            p = jnp.clip(yp.reshape(_CHQ, _R), eps, one - eps)
            y = tm_ref[pl.ds(q * _CHQ, _CHQ), :]
            bce = -(y * jnp.log(p) + (one - y) * jnp.log(one - p))
            acc_ref[...] += bce * vm

        @pl.when(g == _GTC - 1)
        def _():
            out_ref[0, 0] = jnp.sum(acc_ref[...])
            out_ref[0, 1] = jnp.sum((cls_ref[...] > 0).astype(jnp.float32))

    return pl.pallas_call(
        body,
        grid=(_GTC,),
        in_specs=[
            pl.BlockSpec((_B, _R), lambda g: (0, 0)),
            pl.BlockSpec((_ROWS, _R), lambda g: (g, 0)),
            pl.BlockSpec((_QR, _R), lambda g: (g, 0)),
        ],
        out_specs=pl.BlockSpec(memory_space=pltpu.SMEM),
        out_shape=jax.ShapeDtypeStruct((1, 2), jnp.float32),
        scratch_shapes=[
            pltpu.VMEM((_CHR, _R), jnp.float32),
            pltpu.VMEM((_CHQ, _R), jnp.float32),
            pltpu.VMEM((_CHQ, _R), jnp.float32),
        ],
    )(target_class_ids, pred_v, tm_v)


def kernel(target_masks, target_class_ids, pred_masks):
    pred_v = jnp.transpose(pred_masks, (2, 3, 4, 0, 1)).reshape(_HW * _CB, _R)
    tm_v = jnp.transpose(target_masks, (2, 3, 0, 1)).reshape(_HW * _B, _R)

    sc_out = _sc_partial(pred_v, tm_v, target_class_ids.reshape(-1))
    tc_out = _tc_partial(target_class_ids, pred_v, tm_v)

    total = tc_out[0, 0] + jnp.sum(sc_out)
    cnt = tc_out[0, 1]
    return jnp.where(cnt > 0, total / (cnt * jnp.float32(_HW)), jnp.float32(0.0))


# speedup vs baseline: 1.4658x; 1.4658x over previous
import jax
import jax.numpy as jnp
from jax.experimental import pallas as pl
from jax.experimental.pallas import tpu as pltpu

_B, _R = 4, 100
_HW = 784
_C = 81
_PPB = 112
_G = _HW // _PPB
_CB = _C * _B
_ROWS = _PPB * _CB
_QR = _PPB * _B
_PCH = 2
_CHR = _PCH * _CB
_CHQ = _PCH * _B


def _loss_kernel(cls_ref, pred_ref, tm_ref, out_ref, oh_ref, vm_ref, acc_ref):
    g = pl.program_id(0)

    @pl.when(g == 0)
    def _():
        cls = cls_ref[...]
        cid = jax.lax.broadcasted_iota(jnp.int32, (_C, _B, _R), 0)
        oh1 = (cid == cls[None, :, :]).astype(jnp.float32).reshape(_CB, _R)
        vm1 = (cls > 0).astype(jnp.float32)
        for q in range(_PCH):
            oh_ref[pl.ds(q * _CB, _CB), :] = oh1
            vm_ref[pl.ds(q * _B, _B), :] = vm1
        acc_ref[...] = jnp.zeros((_CHQ, _R), jnp.float32)

    eps = jnp.float32(1e-7)
    one = jnp.float32(1.0)
    oh = oh_ref[...]
    vm = vm_ref[...]
    for q in range(_PPB // _PCH):
        xm = pred_ref[pl.ds(q * _CHR, _CHR), :] * oh
        yp = jnp.sum(xm.reshape(_PCH, _C, _B, _R), axis=1)
        p = jnp.clip(yp.reshape(_CHQ, _R), eps, one - eps)
        y = tm_ref[pl.ds(q * _CHQ, _CHQ), :]
        bce = -(y * jnp.log(p) + (one - y) * jnp.log(one - p))
        acc_ref[...] += bce * vm

    @pl.when(g == _G - 1)
    def _():
        cnt = jnp.sum((cls_ref[...] > 0).astype(jnp.float32))
        denom = cnt * jnp.float32(_HW)
        total = jnp.sum(acc_ref[...])
        out_ref[0, 0] = jnp.where(cnt > 0, total / denom, jnp.float32(0.0))


def kernel(target_masks, target_class_ids, pred_masks):
    pred_v = jnp.transpose(pred_masks, (2, 3, 4, 0, 1)).reshape(_HW * _CB, _R)
    tm_v = jnp.transpose(target_masks, (2, 3, 0, 1)).reshape(_HW * _B, _R)

    loss = pl.pallas_call(
        _loss_kernel,
        grid=(_G,),
        in_specs=[
            pl.BlockSpec((_B, _R), lambda g: (0, 0)),
            pl.BlockSpec((_ROWS, _R), lambda g: (g, 0)),
            pl.BlockSpec((_QR, _R), lambda g: (g, 0)),
        ],
        out_specs=pl.BlockSpec(memory_space=pltpu.SMEM),
        out_shape=jax.ShapeDtypeStruct((1, 1), jnp.float32),
        scratch_shapes=[
            pltpu.VMEM((_CHR, _R), jnp.float32),
            pltpu.VMEM((_CHQ, _R), jnp.float32),
            pltpu.VMEM((_CHQ, _R), jnp.float32),
        ],
    )(target_class_ids, pred_v, tm_v)
    return loss[0, 0]


# speedup vs baseline: 1.5454x; 1.0544x over previous
import jax
import jax.numpy as jnp
from jax.experimental import pallas as pl
from jax.experimental.pallas import tpu as pltpu

_B, _R = 4, 100
_HW = 784
_C = 81
_PPB = 112
_G = _HW // _PPB
_CB = _C * _B
_ROWS = _PPB * _CB
_QR = _PPB * _B
_PCH = 2
_CHR = _PCH * _CB
_CHQ = _PCH * _B


def _loss_kernel(cls_ref, pred_a_ref, pred_b_ref, tm_ref, out_ref, oh_ref, vm_ref, acc_ref):
    g = pl.program_id(0)

    @pl.when(g == 0)
    def _():
        cls = cls_ref[...]
        cid = jax.lax.broadcasted_iota(jnp.int32, (_C, _B, _R), 0)
        oh1 = (cid == cls[None, :, :]).astype(jnp.float32).reshape(_CB, _R)
        vm1 = (cls > 0).astype(jnp.float32)
        for q in range(_PCH):
            oh_ref[pl.ds(q * _CB, _CB), :] = oh1
            vm_ref[pl.ds(q * _B, _B), :] = vm1
        acc_ref[...] = jnp.zeros((_CHQ, _R), jnp.float32)

    eps = jnp.float32(1e-7)
    one = jnp.float32(1.0)
    oh = oh_ref[...]
    vm = vm_ref[...]
    half = _PPB // _PCH // 2
    for q in range(_PPB // _PCH):
        pref = pred_a_ref if q < half else pred_b_ref
        qq = q if q < half else q - half
        xm = pref[pl.ds(qq * _CHR, _CHR), :] * oh
        yp = jnp.sum(xm.reshape(_PCH, _C, _B, _R), axis=1)
        p = jnp.clip(yp.reshape(_CHQ, _R), eps, one - eps)
        y = tm_ref[pl.ds(q * _CHQ, _CHQ), :]
        bce = -(y * jnp.log(p) + (one - y) * jnp.log(one - p))
        acc_ref[...] += bce * vm

    @pl.when(g == _G - 1)
    def _():
        cnt = jnp.sum((cls_ref[...] > 0).astype(jnp.float32))
        denom = cnt * jnp.float32(_HW)
        total = jnp.sum(acc_ref[...])
        out_ref[0, 0] = jnp.where(cnt > 0, total / denom, jnp.float32(0.0))


def kernel(target_masks, target_class_ids, pred_masks):
    pred_v = jnp.transpose(pred_masks, (2, 3, 4, 0, 1)).reshape(_HW * _CB, _R)
    tm_v = jnp.transpose(target_masks, (2, 3, 0, 1)).reshape(_HW * _B, _R)

    loss = pl.pallas_call(
        _loss_kernel,
        grid=(_G,),
        in_specs=[
            pl.BlockSpec((_B, _R), lambda g: (0, 0)),
            pl.BlockSpec((_ROWS // 2, _R), lambda g: (2 * g, 0)),
            pl.BlockSpec((_ROWS // 2, _R), lambda g: (2 * g + 1, 0)),
            pl.BlockSpec((_QR, _R), lambda g: (g, 0)),
        ],
        out_specs=pl.BlockSpec(memory_space=pltpu.SMEM),
        out_shape=jax.ShapeDtypeStruct((1, 1), jnp.float32),
        scratch_shapes=[
            pltpu.VMEM((_CHR, _R), jnp.float32),
            pltpu.VMEM((_CHQ, _R), jnp.float32),
            pltpu.VMEM((_CHQ, _R), jnp.float32),
        ],
    )(target_class_ids, pred_v, pred_v, tm_v)
    return loss[0, 0]
